# baseline (speedup 1.0000x reference)
import functools

import jax
import jax.numpy as jnp
from jax import lax
from jax.experimental import pallas as pl
from jax.experimental.pallas import tpu as pltpu
from jax.experimental.pallas import tpu_sc as plsc

F32 = jnp.float32
_B, _P, _DIM, _HEADS = 2, 4096, 128, 8
_KNN = (32, 48, 64)


def _ln_math(x, g, b, eps=1e-5):
    m = jnp.mean(x, axis=-1, keepdims=True)
    v = jnp.mean((x - m) * (x - m), axis=-1, keepdims=True)
    return (x - m) / jnp.sqrt(v + eps) * g + b


def _pick_window(n, c, esize):
    w = 128
    while w * 2 * c * esize * 2 <= 393216 and n % (w * 2) == 0:
        w *= 2
    return w


def _sc_gather(table, idx):
    n = idx.shape[0]
    r, c = table.shape
    assert n % 256 == 0, (n, c)
    window = _pick_window(n, c, table.dtype.itemsize)
    mesh = plsc.VectorSubcoreMesh(core_axis_name="c", subcore_axis_name="s")
    idx2 = idx.reshape(1, n)

    @functools.partial(
        pl.kernel,
        mesh=mesh,
        out_type=jax.ShapeDtypeStruct((n, c), table.dtype),
    )
    def k(table_hbm, idx_hbm, out_hbm):
        def body(i_vmem, o_vmem):
            pltpu.sync_copy(table_hbm.at[i_vmem.at[0]], o_vmem)

        pltpu.emit_pipeline(
            body,
            grid=(n // window,),
            in_specs=[pl.BlockSpec((1, window), lambda i: (0, i))],
            out_specs=[pl.BlockSpec((window, c), lambda i: (i, 0))],
            core_axis_name=("c", "s"),
            dimension_semantics=(pltpu.PARALLEL,),
        )(idx_hbm, out_hbm)

    return k(table, idx2)


def _knn_body(xq_ref, xr_ref, idx_ref, d_ref, keys_ref, *, kk, pr, tq):
    b = pl.program_id(0)
    q = xq_ref[0]
    r = xr_ref[0]
    nch = pr // 128
    a2 = jnp.sum(q * q, axis=1, keepdims=True)
    b2 = jnp.sum(r * r, axis=1)[None, :]
    sq = a2 + b2 - 2.0 * lax.dot_general(
        q, r, (((1,), (1,)), ((), ())), preferred_element_type=F32
    )
    dcl = jnp.maximum(sq, 1e-16)
    bits = lax.bitcast_convert_type(dcl, jnp.int32)
    iota = lax.broadcasted_iota(jnp.int32, (tq, pr), 1)
    keys = (bits & jnp.int32(-4096)) | iota
    keys_ref[...] = keys
    m0 = jnp.min(keys.reshape(tq, nch, 128), axis=2)
    kiota = lax.broadcasted_iota(jnp.int32, (tq, kk), 1)

    def body(t, cm):
        m = jnp.min(cm, axis=1, keepdims=True)
        amin = m & jnp.int32(4095)
        dval = lax.bitcast_convert_type(m & jnp.int32(-4096), F32)
        idx_ref[0] = jnp.where(kiota == t, amin + b * pr, idx_ref[0])
        d_ref[0] = jnp.where(kiota == t, jnp.sqrt(dval), d_ref[0])
        k2 = jnp.where(keys_ref[...] == m, jnp.int32(0x7FFFFFFF), keys_ref[...])
        keys_ref[...] = k2
        return jnp.min(k2.reshape(tq, nch, 128), axis=2)

    lax.fori_loop(0, kk, body, m0)


def _knn(xyz_q, xyz_r, kk, tq=256):
    b, pq, _ = xyz_q.shape
    pr = xyz_r.shape[1]
    tq = min(tq, pq)
    grid = (b, pq // tq)
    out = pl.pallas_call(
        functools.partial(_knn_body, kk=kk, pr=pr, tq=tq),
        grid=grid,
        in_specs=[
            pl.BlockSpec((1, tq, 3), lambda b_, t: (b_, t, 0)),
            pl.BlockSpec((1, pr, 3), lambda b_, t: (b_, 0, 0)),
        ],
        out_specs=[
            pl.BlockSpec((1, tq, kk), lambda b_, t: (b_, t, 0)),
            pl.BlockSpec((1, tq, kk), lambda b_, t: (b_, t, 0)),
        ],
        out_shape=[
            jax.ShapeDtypeStruct((b, pq, kk), jnp.int32),
            jax.ShapeDtypeStruct((b, pq, kk), F32),
        ],
        scratch_shapes=[pltpu.VMEM((tq, pr), jnp.int32)],
    )(xyz_q, xyz_r)
    return out[0], out[1]


def _linear_body(x_ref, w_ref, b_ref, o_ref, *, relu):
    y = jnp.dot(x_ref[...], w_ref[...], preferred_element_type=F32) + b_ref[...]
    if relu:
        y = jnp.maximum(y, 0.0)
    o_ref[...] = y


def _linear(x, w, bias, relu=False, tq=512):
    n, din = x.shape
    dout = w.shape[1]
    tq = min(tq, n)
    out = pl.pallas_call(
        functools.partial(_linear_body, relu=relu),
        grid=(n // tq,),
        in_specs=[
            pl.BlockSpec((tq, din), lambda t: (t, 0)),
            pl.BlockSpec((din, dout), lambda t: (0, 0)),
            pl.BlockSpec((1, dout), lambda t: (0, 0)),
        ],
        out_specs=pl.BlockSpec((tq, dout), lambda t: (t, 0)),
        out_shape=jax.ShapeDtypeStruct((n, dout), F32),
    )(x, w, bias.reshape(1, dout))
    return out


U32 = jnp.uint32
BF16 = jnp.bfloat16


def _dotb(a, b):
    return jnp.dot(a.astype(BF16), b.astype(BF16), preferred_element_type=F32)


def _bf16_bits_rne(x):
    b = lax.bitcast_convert_type(x, U32)
    return (b + jnp.uint32(0x7FFF) + ((b >> 16) & jnp.uint32(1))) >> 16


def _pack_pair(lo, hi):
    w = (_bf16_bits_rne(hi) << 16) | _bf16_bits_rne(lo)
    return lax.bitcast_convert_type(w, jnp.int32)


def _unpack_lo(w_i32):
    w = lax.bitcast_convert_type(w_i32, U32)
    return lax.bitcast_convert_type(w << 16, F32)


def _unpack_hi(w_i32):
    w = lax.bitcast_convert_type(w_i32, U32)
    return lax.bitcast_convert_type(w & jnp.uint32(0xFFFF0000), F32)


def _qkv_body(x_ref, g_ref, b_ref, wq_ref, wk_ref, wv_ref, *out_refs, packed):
    d = x_ref.shape[1]
    h = _ln_math(x_ref[...], g_ref[...], b_ref[...])
    out_refs[0][...] = _dotb(h, wq_ref[...])
    kf = _dotb(h, wk_ref[...])
    vf = _dotb(h, wv_ref[...])
    if packed:
        out_refs[1][...] = _pack_pair(kf, vf)
    else:
        hd = d // 2
        out_refs[1][...] = _pack_pair(kf[:, :hd], kf[:, hd:])
        out_refs[2][...] = _pack_pair(vf[:, :hd], vf[:, hd:])


def _qkv(x, p, tq=256):
    n, d = x.shape
    packed = d <= 256
    tq = min(tq, n)
    wspec = pl.BlockSpec((d, d), lambda t: (0, 0))
    bspec = pl.BlockSpec((1, d), lambda t: (0, 0))
    if packed:
        kv_specs = [pl.BlockSpec((tq, d), lambda t: (t, 0))]
        kv_shapes = [jax.ShapeDtypeStruct((n, d), jnp.int32)]
    else:
        kv_specs = [pl.BlockSpec((tq, d // 2), lambda t: (t, 0))] * 2
        kv_shapes = [jax.ShapeDtypeStruct((n, d // 2), jnp.int32)] * 2
    outs = pl.pallas_call(
        functools.partial(_qkv_body, packed=packed),
        grid=(n // tq,),
        in_specs=[pl.BlockSpec((tq, d), lambda t: (t, 0)), bspec, bspec]
        + [wspec] * 3,
        out_specs=[pl.BlockSpec((tq, d), lambda t: (t, 0))] + kv_specs,
        out_shape=[jax.ShapeDtypeStruct((n, d), F32)] + kv_shapes,
    )(
        x,
        p["ln1_g"].reshape(1, d),
        p["ln1_b"].reshape(1, d),
        p["Wq"],
        p["Wk"],
        p["Wv"],
    )
    return outs[0], outs[1:]


def _attn_body(
    x_ref,
    q_ref,
    xyz_ref,
    xyznei_ref,
    *rest,
    kk,
    heads,
    tq,
    nck,
):
    kv_refs = rest[:nck]
    (w1_ref, b1_ref, w2_ref, b2_ref, wp_ref, bp_ref, o_ref) = rest[nck:]
    d = x_ref.shape[1]
    dh = d // heads
    rows = tq * kk
    relp = xyznei_ref[:, :3] - jnp.broadcast_to(
        xyz_ref[...][:, None, :], (tq, kk, 3)
    ).reshape(rows, 3)
    h1 = jnp.maximum(
        jnp.dot(relp, w1_ref[...], preferred_element_type=F32) + b1_ref[...], 0.0
    )
    pb = _dotb(h1, w2_ref[...]) + b2_ref[...]
    if nck == 1:
        w = kv_refs[0][...]
        k_nei = _unpack_lo(w)
        v_nei = _unpack_hi(w)
    else:
        wk = kv_refs[0][...]
        wv = kv_refs[1][...]
        k_nei = jnp.concatenate([_unpack_lo(wk), _unpack_hi(wk)], axis=1)
        v_nei = jnp.concatenate([_unpack_lo(wv), _unpack_hi(wv)], axis=1)
    qb = jnp.broadcast_to(q_ref[...][:, None, :], (tq, kk, d)).reshape(rows, d)
    hmask = (
        lax.broadcasted_iota(jnp.int32, (d, heads), 0) // dh
        == lax.broadcasted_iota(jnp.int32, (d, heads), 1)
    ).astype(F32)
    hmask_t = (
        lax.broadcasted_iota(jnp.int32, (heads, d), 1) // dh
        == lax.broadcasted_iota(jnp.int32, (heads, d), 0)
    ).astype(F32)
    scale = dh**-0.5
    logits = (
        jnp.dot(qb * (k_nei + pb), hmask, preferred_element_type=F32) * scale
    ).reshape(tq, kk, heads)
    m = jnp.max(logits, axis=1, keepdims=True)
    e = jnp.exp(logits - m)
    s = jnp.sum(e, axis=1, keepdims=True)
    alpha = (e / s).reshape(rows, heads)
    alpha_d = jnp.dot(alpha, hmask_t, preferred_element_type=F32)
    outv = jnp.sum((alpha_d * (v_nei + pb)).reshape(tq, kk, d), axis=1)
    y = _dotb(outv, wp_ref[...]) + bp_ref[...]
    o_ref[...] = x_ref[...] + y


def _attn(x, q, xyz2, xyznei, kvnei_chunks, p, kk, heads, tq):
    n, d = x.shape
    nck = len(kvnei_chunks)
    cw = kvnei_chunks[0].shape[1]
    grid = (n // tq,)
    out = pl.pallas_call(
        functools.partial(_attn_body, kk=kk, heads=heads, tq=tq, nck=nck),
        grid=grid,
        in_specs=[
            pl.BlockSpec((tq, d), lambda t: (t, 0)),
            pl.BlockSpec((tq, d), lambda t: (t, 0)),
            pl.BlockSpec((tq, 3), lambda t: (t, 0)),
            pl.BlockSpec((tq * kk, 128), lambda t: (t, 0)),
        ]
        + [pl.BlockSpec((tq * kk, cw), lambda t: (t, 0))] * nck
        + [
            pl.BlockSpec((3, d), lambda t: (0, 0)),
            pl.BlockSpec((1, d), lambda t: (0, 0)),
            pl.BlockSpec((d, d), lambda t: (0, 0)),
            pl.BlockSpec((1, d), lambda t: (0, 0)),
            pl.BlockSpec((d, d), lambda t: (0, 0)),
            pl.BlockSpec((1, d), lambda t: (0, 0)),
        ],
        out_specs=pl.BlockSpec((tq, d), lambda t: (t, 0)),
        out_shape=jax.ShapeDtypeStruct((n, d), F32),
    )(
        x,
        q,
        xyz2,
        xyznei,
        *kvnei_chunks,
        p["pos1"]["W"],
        p["pos1"]["b"].reshape(1, d),
        p["pos2"]["W"],
        p["pos2"]["b"].reshape(1, d),
        p["proj"]["W"],
        p["proj"]["b"].reshape(1, d),
    )
    return out


def _mlp_body(x_ref, g_ref, b_ref, w1_ref, b1_ref, w2_ref, b2_ref, o_ref):
    x = x_ref[...]
    h = _ln_math(x, g_ref[...], b_ref[...])
    a = _dotb(h, w1_ref[...]) + b1_ref[...]
    ge = a * 0.5 * (1.0 + lax.erf(a * (2.0**-0.5)))
    o_ref[...] = x + _dotb(ge, w2_ref[...]) + b2_ref[...]


def _mlp(x, p, tq=256):
    n, d = x.shape
    dh = p["mlp1"]["W"].shape[1]
    tq = min(tq, n)
    out = pl.pallas_call(
        _mlp_body,
        grid=(n // tq,),
        in_specs=[
            pl.BlockSpec((tq, d), lambda t: (t, 0)),
            pl.BlockSpec((1, d), lambda t: (0, 0)),
            pl.BlockSpec((1, d), lambda t: (0, 0)),
            pl.BlockSpec((d, dh), lambda t: (0, 0)),
            pl.BlockSpec((1, dh), lambda t: (0, 0)),
            pl.BlockSpec((dh, d), lambda t: (0, 0)),
            pl.BlockSpec((1, d), lambda t: (0, 0)),
        ],
        out_specs=pl.BlockSpec((tq, d), lambda t: (t, 0)),
        out_shape=jax.ShapeDtypeStruct((n, d), F32),
    )(
        x,
        p["ln2_g"].reshape(1, d),
        p["ln2_b"].reshape(1, d),
        p["mlp1"]["W"],
        p["mlp1"]["b"].reshape(1, dh),
        p["mlp2"]["W"],
        p["mlp2"]["b"].reshape(1, d),
    )
    return out


def _interp_body(d_ref, nei_ref, skip_ref, wi_ref, ws_ref, b_ref, o_ref, *, tq):
    dl = nei_ref.shape[1]
    kd = jnp.maximum(d_ref[0], 1e-8)
    w = 1.0 / kd
    w = w / jnp.sum(w, axis=-1, keepdims=True)
    nei = nei_ref[...].reshape(tq, 3, dl)
    interp = jnp.sum(w[:, :, None] * nei, axis=1)
    y = jnp.dot(interp, wi_ref[...], preferred_element_type=F32)
    y = y + jnp.dot(skip_ref[...], ws_ref[...], preferred_element_type=F32)
    o_ref[...] = y + b_ref[...]


def _interp(knn_d, nei, skip, w, bias, tq=256):
    b, ph, _ = knn_d.shape
    n = b * ph
    dl = nei.shape[1]
    ds = skip.shape[1]
    do = w.shape[1]
    wi = w[:dl]
    ws = w[dl:]
    kd = knn_d.reshape(1, n, 3)
    out = pl.pallas_call(
        functools.partial(_interp_body, tq=tq),
        grid=(n // tq,),
        in_specs=[
            pl.BlockSpec((1, tq, 3), lambda t: (0, t, 0)),
            pl.BlockSpec((tq * 3, dl), lambda t: (t, 0)),
            pl.BlockSpec((tq, ds), lambda t: (t, 0)),
            pl.BlockSpec((dl, do), lambda t: (0, 0)),
            pl.BlockSpec((ds, do), lambda t: (0, 0)),
            pl.BlockSpec((1, do), lambda t: (0, 0)),
        ],
        out_specs=pl.BlockSpec((tq, do), lambda t: (t, 0)),
        out_shape=jax.ShapeDtypeStruct((n, do), F32),
    )(kd, nei, skip, wi, ws, bias.reshape(1, do))
    return out


def _head_body(x_ref, g_ref, b_ref, w1_ref, b1_ref, w2_ref, b2_ref, o_ref):
    h = _ln_math(x_ref[...], g_ref[...], b_ref[...])
    h = jnp.maximum(
        jnp.dot(h, w1_ref[...], preferred_element_type=F32) + b1_ref[...], 0.0
    )
    o_ref[...] = jnp.dot(h, w2_ref[...], preferred_element_type=F32) + b2_ref[...]


def _head(x, params, tq=512):
    n, d = x.shape
    nc = params["head2"]["W"].shape[1]
    out = pl.pallas_call(
        _head_body,
        grid=(n // tq,),
        in_specs=[
            pl.BlockSpec((tq, d), lambda t: (t, 0)),
            pl.BlockSpec((1, d), lambda t: (0, 0)),
            pl.BlockSpec((1, d), lambda t: (0, 0)),
            pl.BlockSpec((d, d), lambda t: (0, 0)),
            pl.BlockSpec((1, d), lambda t: (0, 0)),
            pl.BlockSpec((d, nc), lambda t: (0, 0)),
            pl.BlockSpec((1, nc), lambda t: (0, 0)),
        ],
        out_specs=pl.BlockSpec((tq, nc), lambda t: (t, 0)),
        out_shape=jax.ShapeDtypeStruct((n, nc), F32),
    )(
        x,
        params["head_ln_g"].reshape(1, d),
        params["head_ln_b"].reshape(1, d),
        params["head1"]["W"],
        params["head1"]["b"].reshape(1, d),
        params["head2"]["W"],
        params["head2"]["b"].reshape(1, nc),
    )
    return out


_CH = 1


def _tf_block(x, xyz2, xyznei_c, idx_c, p, kk, heads, tq_attn):
    n = x.shape[0]
    nch = n // _CH
    q, kv_chunks = _qkv(x, p)
    outs = []
    for c in range(_CH):
        kvnei = [_sc_gather(kc, idx_c[c]) for kc in kv_chunks]
        sl = slice(c * nch, (c + 1) * nch)
        outs.append(
            _attn(x[sl], q[sl], xyz2[sl], xyznei_c[c], kvnei, p, kk, heads, tq_attn)
        )
    x = jnp.concatenate(outs, axis=0)
    x = _mlp(x, p)
    return x


def _stage(x, xyz, blocks, kk, heads, tq_attn):
    b, pp, _ = xyz.shape
    xyz2 = xyz.reshape(b * pp, 3)
    idx, _ = _knn(xyz, xyz, kk)
    idx_c = list(idx.reshape(_CH, b * pp * kk // _CH))
    xyz_pad = jnp.pad(xyz2, ((0, 0), (0, 125)))
    xyznei_c = [_sc_gather(xyz_pad, ic) for ic in idx_c]
    for bp in blocks:
        x = _tf_block(x, xyz2, xyznei_c, idx_c, bp, kk, heads, tq_attn)
    return x


def _downsample(p, xyz, x):
    b, pp, _ = xyz.shape
    m = max(1, pp // 4)
    idx = jnp.linspace(0, pp - 1, m).astype(jnp.int32)
    xyz_d = xyz[:, idx, :]
    d = x.shape[1]
    x_sub = x.reshape(b, pp, d)[:, idx, :].reshape(b * m, d)
    return xyz_d, _linear(x_sub, p["W"], p["b"], tq=256)


def _upsample(p, xyz_low, xyz_high, feat_low, skip):
    idx3, d3 = _knn(xyz_high, xyz_low, 3)
    idxf = idx3.reshape(-1)
    dl = feat_low.shape[1]
    if dl > 256:
        nei = jnp.concatenate(
            [
                _sc_gather(feat_low[:, i * 256 : (i + 1) * 256], idxf)
                for i in range(dl // 256)
            ],
            axis=1,
        )
    else:
        nei = _sc_gather(feat_low, idxf)
    return _interp(d3, nei, skip, p["W"], p["b"])


def kernel(xyz, params):
    b, pp, _ = xyz.shape
    heads = _HEADS
    xyzf = xyz.reshape(b * pp, 3)

    x1 = _linear(xyzf, params["embed"]["W"], params["embed"]["b"])
    x1 = _stage(x1, xyz, params["stage1"], _KNN[0], heads, tq_attn=32)

    xyz2, x2 = _downsample(params["down1"], xyz, x1)
    x2 = _stage(x2, xyz2, params["stage2"], _KNN[1], heads, tq_attn=16)

    xyz3, x3 = _downsample(params["down2"], xyz2, x2)
    x3 = _stage(x3, xyz3, params["stage3"], _KNN[2], heads, tq_attn=16)

    up2 = _upsample(params["up2"], xyz3, xyz2, x3, x2)
    up1 = _upsample(params["up1"], xyz2, xyz, up2, x1)

    out = _head(up1, params)
    return out.reshape(b, pp, params["head2"]["W"].shape[1])


# speedup vs baseline: 2.1110x; 2.1110x over previous
import functools

import jax
import jax.numpy as jnp
from jax import lax
from jax.experimental import pallas as pl
from jax.experimental.pallas import tpu as pltpu
from jax.experimental.pallas import tpu_sc as plsc

F32 = jnp.float32
_B, _P, _DIM, _HEADS = 2, 4096, 128, 8
_KNN = (32, 48, 64)


def _ln_math(x, g, b, eps=1e-5):
    m = jnp.mean(x, axis=-1, keepdims=True)
    v = jnp.mean((x - m) * (x - m), axis=-1, keepdims=True)
    return (x - m) / jnp.sqrt(v + eps) * g + b


def _pick_window(n, c, esize):
    w = 128
    while w * 2 * c * esize * 2 <= 393216 and n % (w * 2) == 0:
        w *= 2
    return w


def _sc_gather(table, idx):
    n = idx.shape[0]
    r, c = table.shape
    assert n % 256 == 0, (n, c)
    window = _pick_window(n, c, table.dtype.itemsize)
    mesh = plsc.VectorSubcoreMesh(core_axis_name="c", subcore_axis_name="s")
    idx2 = idx.reshape(1, n)

    @functools.partial(
        pl.kernel,
        mesh=mesh,
        out_type=jax.ShapeDtypeStruct((n, c), table.dtype),
    )
    def k(table_hbm, idx_hbm, out_hbm):
        def body(i_vmem, o_vmem):
            pltpu.sync_copy(table_hbm.at[i_vmem.at[0]], o_vmem)

        pltpu.emit_pipeline(
            body,
            grid=(n // window,),
            in_specs=[pl.BlockSpec((1, window), lambda i: (0, i))],
            out_specs=[pl.BlockSpec((window, c), lambda i: (i, 0))],
            core_axis_name=("c", "s"),
            dimension_semantics=(pltpu.PARALLEL,),
        )(idx_hbm, out_hbm)

    return k(table, idx2)


def _knn_body(xq_ref, xr_ref, idx_ref, d_ref, keys_ref, *, kk, pr, tq):
    b = pl.program_id(0)
    q = xq_ref[0]
    r = xr_ref[0]
    nch = pr // 128
    a2 = jnp.sum(q * q, axis=1, keepdims=True)
    b2 = jnp.sum(r * r, axis=1)[None, :]
    sq = a2 + b2 - 2.0 * lax.dot_general(
        q, r, (((1,), (1,)), ((), ())), preferred_element_type=F32
    )
    dcl = jnp.maximum(sq, 1e-16)
    bits = lax.bitcast_convert_type(dcl, jnp.int32)
    iota = lax.broadcasted_iota(jnp.int32, (tq, pr), 1)
    keys = (bits & jnp.int32(-4096)) | iota
    keys_ref[...] = keys
    kiota = lax.broadcasted_iota(jnp.int32, (tq, kk), 1)

    def body(t, _):
        ks = keys_ref[...]
        m = jnp.min(ks, axis=1, keepdims=True)
        amin = m & jnp.int32(4095)
        dval = lax.bitcast_convert_type(m & jnp.int32(-4096), F32)
        idx_ref[0] = jnp.where(kiota == t, amin + b * pr, idx_ref[0])
        d_ref[0] = jnp.where(kiota == t, jnp.sqrt(dval), d_ref[0])
        keys_ref[...] = jnp.where(ks == m, jnp.int32(0x7FFFFFFF), ks)
        return 0

    lax.fori_loop(0, kk, body, 0)


def _knn(xyz_q, xyz_r, kk, tq=256):
    b, pq, _ = xyz_q.shape
    pr = xyz_r.shape[1]
    tq = min(tq, pq)
    grid = (b, pq // tq)
    out = pl.pallas_call(
        functools.partial(_knn_body, kk=kk, pr=pr, tq=tq),
        grid=grid,
        in_specs=[
            pl.BlockSpec((1, tq, 3), lambda b_, t: (b_, t, 0)),
            pl.BlockSpec((1, pr, 3), lambda b_, t: (b_, 0, 0)),
        ],
        out_specs=[
            pl.BlockSpec((1, tq, kk), lambda b_, t: (b_, t, 0)),
            pl.BlockSpec((1, tq, kk), lambda b_, t: (b_, t, 0)),
        ],
        out_shape=[
            jax.ShapeDtypeStruct((b, pq, kk), jnp.int32),
            jax.ShapeDtypeStruct((b, pq, kk), F32),
        ],
        scratch_shapes=[pltpu.VMEM((tq, pr), jnp.int32)],
    )(xyz_q, xyz_r)
    return out[0], out[1]


def _linear_body(x_ref, w_ref, b_ref, o_ref, *, relu):
    y = jnp.dot(x_ref[...], w_ref[...], preferred_element_type=F32) + b_ref[...]
    if relu:
        y = jnp.maximum(y, 0.0)
    o_ref[...] = y


def _linear(x, w, bias, relu=False, tq=512):
    n, din = x.shape
    dout = w.shape[1]
    tq = min(tq, n)
    out = pl.pallas_call(
        functools.partial(_linear_body, relu=relu),
        grid=(n // tq,),
        in_specs=[
            pl.BlockSpec((tq, din), lambda t: (t, 0)),
            pl.BlockSpec((din, dout), lambda t: (0, 0)),
            pl.BlockSpec((1, dout), lambda t: (0, 0)),
        ],
        out_specs=pl.BlockSpec((tq, dout), lambda t: (t, 0)),
        out_shape=jax.ShapeDtypeStruct((n, dout), F32),
    )(x, w, bias.reshape(1, dout))
    return out


U32 = jnp.uint32
BF16 = jnp.bfloat16


def _dotb(a, b):
    return jnp.dot(a.astype(BF16), b.astype(BF16), preferred_element_type=F32)


def _bf16_bits_rne(x):
    b = lax.bitcast_convert_type(x, U32)
    return (b + jnp.uint32(0x7FFF) + ((b >> 16) & jnp.uint32(1))) >> 16


def _pack_pair(lo, hi):
    w = (_bf16_bits_rne(hi) << 16) | _bf16_bits_rne(lo)
    return lax.bitcast_convert_type(w, jnp.int32)


def _unpack_lo(w_i32):
    w = lax.bitcast_convert_type(w_i32, U32)
    return lax.bitcast_convert_type(w << 16, F32)


def _unpack_hi(w_i32):
    w = lax.bitcast_convert_type(w_i32, U32)
    return lax.bitcast_convert_type(w & jnp.uint32(0xFFFF0000), F32)


def _qkv_body(x_ref, g_ref, b_ref, wq_ref, wk_ref, wv_ref, *out_refs, packed):
    d = x_ref.shape[1]
    h = _ln_math(x_ref[...], g_ref[...], b_ref[...])
    out_refs[0][...] = _dotb(h, wq_ref[...])
    kf = _dotb(h, wk_ref[...])
    vf = _dotb(h, wv_ref[...])
    if packed:
        out_refs[1][...] = _pack_pair(kf, vf)
    else:
        hd = d // 2
        out_refs[1][...] = _pack_pair(kf[:, :hd], kf[:, hd:])
        out_refs[2][...] = _pack_pair(vf[:, :hd], vf[:, hd:])


def _qkv(x, p, tq=256):
    n, d = x.shape
    packed = d <= 256
    tq = min(tq, n)
    wspec = pl.BlockSpec((d, d), lambda t: (0, 0))
    bspec = pl.BlockSpec((1, d), lambda t: (0, 0))
    if packed:
        kv_specs = [pl.BlockSpec((tq, d), lambda t: (t, 0))]
        kv_shapes = [jax.ShapeDtypeStruct((n, d), jnp.int32)]
    else:
        kv_specs = [pl.BlockSpec((tq, d // 2), lambda t: (t, 0))] * 2
        kv_shapes = [jax.ShapeDtypeStruct((n, d // 2), jnp.int32)] * 2
    outs = pl.pallas_call(
        functools.partial(_qkv_body, packed=packed),
        grid=(n // tq,),
        in_specs=[pl.BlockSpec((tq, d), lambda t: (t, 0)), bspec, bspec]
        + [wspec] * 3,
        out_specs=[pl.BlockSpec((tq, d), lambda t: (t, 0))] + kv_specs,
        out_shape=[jax.ShapeDtypeStruct((n, d), F32)] + kv_shapes,
    )(
        x,
        p["ln1_g"].reshape(1, d),
        p["ln1_b"].reshape(1, d),
        p["Wq"],
        p["Wk"],
        p["Wv"],
    )
    return outs[0], outs[1:]


def _attn_body(
    x_ref,
    q_ref,
    xyz_ref,
    xyznei_ref,
    *rest,
    kk,
    heads,
    tq,
    nck,
):
    kv_refs = rest[:nck]
    (w1_ref, b1_ref, w2_ref, b2_ref, wp_ref, bp_ref, o_ref) = rest[nck:]
    d = x_ref.shape[1]
    dh = d // heads
    rows = tq * kk
    relp = xyznei_ref[:, :3] - jnp.broadcast_to(
        xyz_ref[...][:, None, :], (tq, kk, 3)
    ).reshape(rows, 3)
    h1 = jnp.maximum(
        jnp.dot(relp, w1_ref[...], preferred_element_type=F32) + b1_ref[...], 0.0
    )
    pb = _dotb(h1, w2_ref[...]) + b2_ref[...]
    if nck == 1:
        w = kv_refs[0][...]
        k_nei = _unpack_lo(w)
        v_nei = _unpack_hi(w)
    else:
        wk = kv_refs[0][...]
        wv = kv_refs[1][...]
        k_nei = jnp.concatenate([_unpack_lo(wk), _unpack_hi(wk)], axis=1)
        v_nei = jnp.concatenate([_unpack_lo(wv), _unpack_hi(wv)], axis=1)
    qb = jnp.broadcast_to(q_ref[...][:, None, :], (tq, kk, d)).reshape(rows, d)
    hmask = (
        lax.broadcasted_iota(jnp.int32, (d, heads), 0) // dh
        == lax.broadcasted_iota(jnp.int32, (d, heads), 1)
    ).astype(F32)
    hmask_t = (
        lax.broadcasted_iota(jnp.int32, (heads, d), 1) // dh
        == lax.broadcasted_iota(jnp.int32, (heads, d), 0)
    ).astype(F32)
    scale = dh**-0.5
    logits = (
        jnp.dot(qb * (k_nei + pb), hmask, preferred_element_type=F32) * scale
    ).reshape(tq, kk, heads)
    m = jnp.max(logits, axis=1, keepdims=True)
    e = jnp.exp(logits - m)
    s = jnp.sum(e, axis=1, keepdims=True)
    alpha = (e / s).reshape(rows, heads)
    alpha_d = jnp.dot(alpha, hmask_t, preferred_element_type=F32)
    outv = jnp.sum((alpha_d * (v_nei + pb)).reshape(tq, kk, d), axis=1)
    y = _dotb(outv, wp_ref[...]) + bp_ref[...]
    o_ref[...] = x_ref[...] + y


def _attn(x, q, xyz2, xyznei, kvnei_chunks, p, kk, heads, tq):
    n, d = x.shape
    nck = len(kvnei_chunks)
    cw = kvnei_chunks[0].shape[1]
    grid = (n // tq,)
    out = pl.pallas_call(
        functools.partial(_attn_body, kk=kk, heads=heads, tq=tq, nck=nck),
        grid=grid,
        in_specs=[
            pl.BlockSpec((tq, d), lambda t: (t, 0)),
            pl.BlockSpec((tq, d), lambda t: (t, 0)),
            pl.BlockSpec((tq, 3), lambda t: (t, 0)),
            pl.BlockSpec((tq * kk, 128), lambda t: (t, 0)),
        ]
        + [pl.BlockSpec((tq * kk, cw), lambda t: (t, 0))] * nck
        + [
            pl.BlockSpec((3, d), lambda t: (0, 0)),
            pl.BlockSpec((1, d), lambda t: (0, 0)),
            pl.BlockSpec((d, d), lambda t: (0, 0)),
            pl.BlockSpec((1, d), lambda t: (0, 0)),
            pl.BlockSpec((d, d), lambda t: (0, 0)),
            pl.BlockSpec((1, d), lambda t: (0, 0)),
        ],
        out_specs=pl.BlockSpec((tq, d), lambda t: (t, 0)),
        out_shape=jax.ShapeDtypeStruct((n, d), F32),
    )(
        x,
        q,
        xyz2,
        xyznei,
        *kvnei_chunks,
        p["pos1"]["W"],
        p["pos1"]["b"].reshape(1, d),
        p["pos2"]["W"],
        p["pos2"]["b"].reshape(1, d),
        p["proj"]["W"],
        p["proj"]["b"].reshape(1, d),
    )
    return out


def _mlp_body(x_ref, g_ref, b_ref, w1_ref, b1_ref, w2_ref, b2_ref, o_ref):
    x = x_ref[...]
    h = _ln_math(x, g_ref[...], b_ref[...])
    a = _dotb(h, w1_ref[...]) + b1_ref[...]
    ge = a * 0.5 * (1.0 + lax.erf(a * (2.0**-0.5)))
    o_ref[...] = x + _dotb(ge, w2_ref[...]) + b2_ref[...]


def _mlp(x, p, tq=256):
    n, d = x.shape
    dh = p["mlp1"]["W"].shape[1]
    tq = min(tq, n)
    out = pl.pallas_call(
        _mlp_body,
        grid=(n // tq,),
        in_specs=[
            pl.BlockSpec((tq, d), lambda t: (t, 0)),
            pl.BlockSpec((1, d), lambda t: (0, 0)),
            pl.BlockSpec((1, d), lambda t: (0, 0)),
            pl.BlockSpec((d, dh), lambda t: (0, 0)),
            pl.BlockSpec((1, dh), lambda t: (0, 0)),
            pl.BlockSpec((dh, d), lambda t: (0, 0)),
            pl.BlockSpec((1, d), lambda t: (0, 0)),
        ],
        out_specs=pl.BlockSpec((tq, d), lambda t: (t, 0)),
        out_shape=jax.ShapeDtypeStruct((n, d), F32),
    )(
        x,
        p["ln2_g"].reshape(1, d),
        p["ln2_b"].reshape(1, d),
        p["mlp1"]["W"],
        p["mlp1"]["b"].reshape(1, dh),
        p["mlp2"]["W"],
        p["mlp2"]["b"].reshape(1, d),
    )
    return out


def _interp_body(d_ref, nei_ref, skip_ref, wi_ref, ws_ref, b_ref, o_ref, *, tq):
    dl = nei_ref.shape[1]
    kd = jnp.maximum(d_ref[0], 1e-8)
    w = 1.0 / kd
    w = w / jnp.sum(w, axis=-1, keepdims=True)
    nei = nei_ref[...].reshape(tq, 3, dl)
    interp = jnp.sum(w[:, :, None] * nei, axis=1)
    y = jnp.dot(interp, wi_ref[...], preferred_element_type=F32)
    y = y + jnp.dot(skip_ref[...], ws_ref[...], preferred_element_type=F32)
    o_ref[...] = y + b_ref[...]


def _interp(knn_d, nei, skip, w, bias, tq=256):
    b, ph, _ = knn_d.shape
    n = b * ph
    dl = nei.shape[1]
    ds = skip.shape[1]
    do = w.shape[1]
    wi = w[:dl]
    ws = w[dl:]
    kd = knn_d.reshape(1, n, 3)
    out = pl.pallas_call(
        functools.partial(_interp_body, tq=tq),
        grid=(n // tq,),
        in_specs=[
            pl.BlockSpec((1, tq, 3), lambda t: (0, t, 0)),
            pl.BlockSpec((tq * 3, dl), lambda t: (t, 0)),
            pl.BlockSpec((tq, ds), lambda t: (t, 0)),
            pl.BlockSpec((dl, do), lambda t: (0, 0)),
            pl.BlockSpec((ds, do), lambda t: (0, 0)),
            pl.BlockSpec((1, do), lambda t: (0, 0)),
        ],
        out_specs=pl.BlockSpec((tq, do), lambda t: (t, 0)),
        out_shape=jax.ShapeDtypeStruct((n, do), F32),
    )(kd, nei, skip, wi, ws, bias.reshape(1, do))
    return out


def _head_body(x_ref, g_ref, b_ref, w1_ref, b1_ref, w2_ref, b2_ref, o_ref):
    h = _ln_math(x_ref[...], g_ref[...], b_ref[...])
    h = jnp.maximum(
        jnp.dot(h, w1_ref[...], preferred_element_type=F32) + b1_ref[...], 0.0
    )
    o_ref[...] = jnp.dot(h, w2_ref[...], preferred_element_type=F32) + b2_ref[...]


def _head(x, params, tq=512):
    n, d = x.shape
    nc = params["head2"]["W"].shape[1]
    out = pl.pallas_call(
        _head_body,
        grid=(n // tq,),
        in_specs=[
            pl.BlockSpec((tq, d), lambda t: (t, 0)),
            pl.BlockSpec((1, d), lambda t: (0, 0)),
            pl.BlockSpec((1, d), lambda t: (0, 0)),
            pl.BlockSpec((d, d), lambda t: (0, 0)),
            pl.BlockSpec((1, d), lambda t: (0, 0)),
            pl.BlockSpec((d, nc), lambda t: (0, 0)),
            pl.BlockSpec((1, nc), lambda t: (0, 0)),
        ],
        out_specs=pl.BlockSpec((tq, nc), lambda t: (t, 0)),
        out_shape=jax.ShapeDtypeStruct((n, nc), F32),
    )(
        x,
        params["head_ln_g"].reshape(1, d),
        params["head_ln_b"].reshape(1, d),
        params["head1"]["W"],
        params["head1"]["b"].reshape(1, d),
        params["head2"]["W"],
        params["head2"]["b"].reshape(1, nc),
    )
    return out


_CH = 1


def _tf_block(x, xyz2, xyznei_c, idx_c, p, kk, heads, tq_attn):
    n = x.shape[0]
    nch = n // _CH
    q, kv_chunks = _qkv(x, p)
    outs = []
    for c in range(_CH):
        kvnei = [_sc_gather(kc, idx_c[c]) for kc in kv_chunks]
        sl = slice(c * nch, (c + 1) * nch)
        outs.append(
            _attn(x[sl], q[sl], xyz2[sl], xyznei_c[c], kvnei, p, kk, heads, tq_attn)
        )
    x = jnp.concatenate(outs, axis=0)
    x = _mlp(x, p)
    return x


def _stage(x, xyz, blocks, kk, heads, tq_attn):
    b, pp, _ = xyz.shape
    xyz2 = xyz.reshape(b * pp, 3)
    idx, _ = _knn(xyz, xyz, kk)
    idx_c = list(idx.reshape(_CH, b * pp * kk // _CH))
    xyz_pad = jnp.pad(xyz2, ((0, 0), (0, 125)))
    xyznei_c = [_sc_gather(xyz_pad, ic) for ic in idx_c]
    for bp in blocks:
        x = _tf_block(x, xyz2, xyznei_c, idx_c, bp, kk, heads, tq_attn)
    return x


def _downsample(p, xyz, x):
    b, pp, _ = xyz.shape
    m = max(1, pp // 4)
    idx = jnp.linspace(0, pp - 1, m).astype(jnp.int32)
    xyz_d = xyz[:, idx, :]
    d = x.shape[1]
    x_sub = x.reshape(b, pp, d)[:, idx, :].reshape(b * m, d)
    return xyz_d, _linear(x_sub, p["W"], p["b"], tq=256)


def _upsample(p, xyz_low, xyz_high, feat_low, skip):
    idx3, d3 = _knn(xyz_high, xyz_low, 3)
    idxf = idx3.reshape(-1)
    dl = feat_low.shape[1]
    if dl > 256:
        nei = jnp.concatenate(
            [
                _sc_gather(feat_low[:, i * 256 : (i + 1) * 256], idxf)
                for i in range(dl // 256)
            ],
            axis=1,
        )
    else:
        nei = _sc_gather(feat_low, idxf)
    return _interp(d3, nei, skip, p["W"], p["b"])


def kernel(xyz, params):
    b, pp, _ = xyz.shape
    heads = _HEADS
    xyzf = xyz.reshape(b * pp, 3)

    x1 = _linear(xyzf, params["embed"]["W"], params["embed"]["b"])
    x1 = _stage(x1, xyz, params["stage1"], _KNN[0], heads, tq_attn=32)

    xyz2, x2 = _downsample(params["down1"], xyz, x1)
    x2 = _stage(x2, xyz2, params["stage2"], _KNN[1], heads, tq_attn=16)

    xyz3, x3 = _downsample(params["down2"], xyz2, x2)
    x3 = _stage(x3, xyz3, params["stage3"], _KNN[2], heads, tq_attn=16)

    up2 = _upsample(params["up2"], xyz3, xyz2, x3, x2)
    up1 = _upsample(params["up1"], xyz2, xyz, up2, x1)

    out = _head(up1, params)
    return out.reshape(b, pp, params["head2"]["W"].shape[1])
